# baseline (speedup 1.0000x reference)
import functools

import jax
import jax.numpy as jnp
from jax import lax
from jax.experimental import pallas as pl
from jax.experimental.pallas import tpu as pltpu
from jax.experimental.pallas import tpu_sc as plsc

N = 10000
E = 320000
DIN = 128
DOUT = 32
NPAD = 10240
NC = 2
NS = 16
NW = NC * NS
RPT = NPAD // NS
CH = 80
NCHUNK = 125
NBUF = 5
NBUF2 = 10
DEG_LAG = 8


def _sc_mesh():
    return plsc.VectorSubcoreMesh(core_axis_name="c", subcore_axis_name="s")


@functools.partial(
    pl.kernel,
    out_type=jax.ShapeDtypeStruct((NC, NPAD), jnp.float32),
    mesh=_sc_mesh(),
    scratch_types=[
        pltpu.VMEM((NCHUNK, CH), jnp.int32),
        pltpu.VMEM((NCHUNK, CH), jnp.float32),
        pltpu.VMEM_SHARED((NPAD,), jnp.float32),
        pltpu.SemaphoreType.DMA,
    ],
)
def _sc_deg(ei4, ew3, zeros, out, dst_all, ew_all, acc, ssem):
    cid = lax.axis_index("c")
    sid = lax.axis_index("s")
    wid = cid * NS + sid
    pltpu.sync_copy(ei4.at[1, wid], dst_all)
    pltpu.sync_copy(ew3.at[wid], ew_all)
    pltpu.sync_copy(zeros.at[pl.ds(sid * RPT, RPT)], acc.at[pl.ds(sid * RPT, RPT)])
    plsc.subcore_barrier()

    def fire(j):
        pltpu.make_async_copy(ew_all.at[j], acc.at[dst_all.at[j]], ssem).start(
            add=True
        )

    def drain(j):
        pltpu.make_async_copy(ew_all.at[j], acc.at[dst_all.at[j]], ssem).wait()

    def body(j, carry):
        fire(j)

        @pl.when(j >= DEG_LAG)
        def _():
            drain(j - DEG_LAG)

        return carry

    lax.fori_loop(0, NCHUNK, body, 0)
    for j in range(NCHUNK - DEG_LAG, NCHUNK):
        drain(j)
    plsc.subcore_barrier()
    pltpu.sync_copy(acc.at[pl.ds(sid * RPT, RPT)], out.at[cid, pl.ds(sid * RPT, RPT)])


def _make_sc_agg(D):
    @functools.partial(
        pl.kernel,
        out_type=jax.ShapeDtypeStruct((NC, NPAD, D), jnp.float32),
        mesh=_sc_mesh(),
        compiler_params=pltpu.CompilerParams(use_tc_tiling_on_sc=False),
        scratch_types=[
            pltpu.VMEM((NCHUNK, CH), jnp.int32),
            pltpu.VMEM((NCHUNK, CH), jnp.int32),
            pltpu.VMEM((NCHUNK, CH), jnp.float32),
        ]
        + [pltpu.VMEM((CH, D), jnp.float32) for _ in range(NBUF)]
        + [pltpu.VMEM_SHARED((NPAD, D), jnp.float32)]
        + [pltpu.SemaphoreType.DMA for _ in range(2 * NBUF)],
    )
    def _sc_agg(ei4, ew3, tbl, zeros, out, src_all, dst_all, ew_all,
                *bufs_acc_sems):
        rows = bufs_acc_sems[:NBUF]
        acc = bufs_acc_sems[NBUF]
        gsems = bufs_acc_sems[NBUF + 1:2 * NBUF + 1]
        ssems = bufs_acc_sems[2 * NBUF + 1:]
        cid = lax.axis_index("c")
        sid = lax.axis_index("s")
        wid = cid * NS + sid
        pltpu.sync_copy(ei4.at[0, wid], src_all)
        pltpu.sync_copy(ei4.at[1, wid], dst_all)
        pltpu.sync_copy(ew3.at[wid], ew_all)
        pltpu.sync_copy(zeros.at[pl.ds(sid * RPT, RPT)], acc.at[pl.ds(sid * RPT, RPT)])
        plsc.subcore_barrier()

        def g_copy(j, b):
            return pltpu.make_async_copy(tbl.at[src_all.at[j]], rows[b], gsems[b])

        def s_copy(j, b):
            return pltpu.make_async_copy(rows[b], acc.at[dst_all.at[j]], ssems[b])

        def scale(b, j):
            def grp(g, c):
                ev = ew_all[j, pl.ds(g * 16, 16)]
                for i in range(16):
                    s = ev[i]
                    for k in range(D // 16):
                        sl = pl.ds(k * 16, 16)
                        rows[b][g * 16 + i, sl] = rows[b][g * 16 + i, sl] * s
                return c

            lax.fori_loop(0, CH // 16, grp, 0)

        for b in range(NBUF):
            g_copy(b, b).start()

        def step(j2, carry):
            j0 = NBUF * j2
            for b in range(NBUF):
                j = j0 + b
                g_copy(j, b).wait()
                scale(b, j)
                s_copy(j, b).start(add=True)

                @pl.when(j + NBUF < NCHUNK)
                def _():
                    s_copy(j, b).wait()
                    g_copy(j + NBUF, b).start()

            return carry

        lax.fori_loop(0, NCHUNK // NBUF, step, 0)
        for b in range(NBUF):
            s_copy(NCHUNK - NBUF + b, b).wait()
        plsc.subcore_barrier()
        pltpu.sync_copy(
            acc.at[pl.ds(sid * RPT, RPT)], out.at[cid, pl.ds(sid * RPT, RPT)]
        )

    return _sc_agg


_sc_agg32 = _make_sc_agg(32)

_DS = 32


@functools.partial(
    pl.kernel,
    out_type=jax.ShapeDtypeStruct((NC, 2, NPAD, _DS), jnp.float32),
    mesh=_sc_mesh(),
    compiler_params=pltpu.CompilerParams(use_tc_tiling_on_sc=False),
    scratch_types=[
        pltpu.VMEM((NCHUNK, CH), jnp.int32),
        pltpu.VMEM((NCHUNK, CH), jnp.int32),
        pltpu.VMEM((NCHUNK, CH), jnp.float32),
    ]
    + [pltpu.VMEM((CH, _DS), jnp.float32) for _ in range(NBUF2)]
    + [
        pltpu.VMEM_SHARED((NPAD, _DS), jnp.float32),
        pltpu.VMEM_SHARED((NPAD, _DS), jnp.float32),
    ]
    + [pltpu.SemaphoreType.DMA for _ in range(2 * NBUF2)],
)
def _sc_agg_split(ei4, ew3, tbla, tblb, zeros, out, src_all, dst_all,
                  ew_all, *bufs_acc_sems):
    rows = bufs_acc_sems[:NBUF2]
    acca = bufs_acc_sems[NBUF2]
    accb = bufs_acc_sems[NBUF2 + 1]
    gsems = bufs_acc_sems[NBUF2 + 2:2 * NBUF2 + 2]
    ssems = bufs_acc_sems[2 * NBUF2 + 2:]
    tbls = (tbla, tblb)
    accs = (acca, accb)
    cid = lax.axis_index("c")
    sid = lax.axis_index("s")
    wid = cid * NS + sid
    pltpu.sync_copy(ei4.at[0, wid], src_all)
    pltpu.sync_copy(ei4.at[1, wid], dst_all)
    pltpu.sync_copy(ew3.at[wid], ew_all)
    sl_n = pl.ds(sid * RPT, RPT)
    pltpu.sync_copy(zeros.at[sl_n], acca.at[sl_n])
    pltpu.sync_copy(zeros.at[sl_n], accb.at[sl_n])
    plsc.subcore_barrier()

    def g_copy(j, b):
        return pltpu.make_async_copy(
            tbls[b & 1].at[src_all.at[j]], rows[b], gsems[b]
        )

    def s_copy(j, b):
        return pltpu.make_async_copy(
            rows[b], accs[b & 1].at[dst_all.at[j]], ssems[b]
        )

    def scale_pair(ba, bb, j):
        def grp(g, c):
            ev = ew_all[j, pl.ds(g * 16, 16)]
            for i in range(16):
                s = ev[i]
                for k in range(_DS // 16):
                    sl = pl.ds(k * 16, 16)
                    rows[ba][g * 16 + i, sl] = rows[ba][g * 16 + i, sl] * s
                    rows[bb][g * 16 + i, sl] = rows[bb][g * 16 + i, sl] * s
            return c

        lax.fori_loop(0, CH // 16, grp, 0)

    for b in range(NBUF2):
        g_copy(b >> 1, b).start()

    NT = 2 * NCHUNK

    def step(t2, carry):
        t0 = NBUF2 * t2
        for p in range(NBUF2 // 2):
            ba, bb = 2 * p, 2 * p + 1
            j = NBUF2 // 2 * t2 + p
            g_copy(j, ba).wait()
            g_copy(j, bb).wait()
            scale_pair(ba, bb, j)
            s_copy(j, ba).start(add=True)
            s_copy(j, bb).start(add=True)

            @pl.when(j + (NBUF2 >> 1) < NCHUNK)
            def _():
                s_copy(j, ba).wait()
                s_copy(j, bb).wait()
                g_copy(j + (NBUF2 >> 1), ba).start()
                g_copy(j + (NBUF2 >> 1), bb).start()

        return carry

    lax.fori_loop(0, NT // NBUF2, step, 0)
    for b in range(NBUF2):
        s_copy(NCHUNK - (NBUF2 >> 1) + (b >> 1), b).wait()
    plsc.subcore_barrier()
    pltpu.sync_copy(acca.at[sl_n], out.at[cid, 0, sl_n])
    pltpu.sync_copy(accb.at[sl_n], out.at[cid, 1, sl_n])


_GB = 10
_RB = N // _GB


def _tc1_body(degp, x, w, dinv_o, xwsa_o, xwsb_o):
    deg = degp[0] + degp[1] + 1.0
    dinv = lax.rsqrt(deg)
    xw = jnp.dot(x[...], w[...], preferred_element_type=jnp.float32)
    xws = xw * dinv
    dinv_o[...] = dinv
    xwsa_o[...] = xws[:, :_DS]
    xwsb_o[...] = xws[:, _DS:]


_tc1 = pl.pallas_call(
    _tc1_body,
    grid=(_GB,),
    in_specs=[
        pl.BlockSpec((2, _RB, 1), lambda i: (0, i, 0)),
        pl.BlockSpec((_RB, DIN), lambda i: (i, 0)),
        pl.BlockSpec((DIN, 2 * _DS), lambda i: (0, 0)),
    ],
    out_specs=[
        pl.BlockSpec((_RB, 1), lambda i: (i, 0)),
        pl.BlockSpec((_RB, _DS), lambda i: (i, 0)),
        pl.BlockSpec((_RB, _DS), lambda i: (i, 0)),
    ],
    out_shape=[
        jax.ShapeDtypeStruct((N, 1), jnp.float32),
        jax.ShapeDtypeStruct((N, _DS), jnp.float32),
        jax.ShapeDtypeStruct((N, _DS), jnp.float32),
    ],
)


def _tc2_body(acc1, xwsa, xwsb, dinv, b1, w2, out):
    ta = (acc1[0, 0] + acc1[1, 0] + xwsa[...]) * dinv[...] + b1[:, :_DS]
    tb = (acc1[0, 1] + acc1[1, 1] + xwsb[...]) * dinv[...] + b1[:, _DS:]
    h = jnp.concatenate([jnp.maximum(ta, 0.0), jnp.maximum(tb, 0.0)], axis=1)
    hw2 = jnp.dot(h, w2[...], preferred_element_type=jnp.float32)
    out[...] = hw2 * dinv[...]


_tc2 = pl.pallas_call(
    _tc2_body,
    grid=(_GB,),
    in_specs=[
        pl.BlockSpec((2, 2, _RB, _DS), lambda i: (0, 0, i, 0)),
        pl.BlockSpec((_RB, _DS), lambda i: (i, 0)),
        pl.BlockSpec((_RB, _DS), lambda i: (i, 0)),
        pl.BlockSpec((_RB, 1), lambda i: (i, 0)),
        pl.BlockSpec((1, 2 * _DS), lambda i: (0, 0)),
        pl.BlockSpec((2 * _DS, DOUT), lambda i: (0, 0)),
    ],
    out_specs=pl.BlockSpec((_RB, DOUT), lambda i: (i, 0)),
    out_shape=jax.ShapeDtypeStruct((N, DOUT), jnp.float32),
)


def _tc3_body(acc2, hw, dinv, b2, out):
    out[...] = (acc2[0] + acc2[1] + hw[...]) * dinv[...] + b2[...]


_tc3 = pl.pallas_call(
    _tc3_body,
    grid=(_GB,),
    in_specs=[
        pl.BlockSpec((2, _RB, DOUT), lambda i: (0, i, 0)),
        pl.BlockSpec((_RB, DOUT), lambda i: (i, 0)),
        pl.BlockSpec((_RB, 1), lambda i: (i, 0)),
        pl.BlockSpec((1, DOUT), lambda i: (0, 0)),
    ],
    out_specs=pl.BlockSpec((_RB, DOUT), lambda i: (i, 0)),
    out_shape=jax.ShapeDtypeStruct((N, DOUT), jnp.float32),
)


def kernel(x, edge_index, edge_weight, W1, b1, W2, b2):
    ei4 = edge_index.astype(jnp.int32).reshape(2, NW, NCHUNK, CH)
    ew3 = edge_weight.reshape(NW, NCHUNK, CH)
    z1 = jnp.zeros((NPAD,), jnp.float32)
    z32 = jnp.zeros((NPAD, 32), jnp.float32)

    degp = _sc_deg(ei4, ew3, z1)
    dinv, xwsa, xwsb = _tc1(degp.reshape(2, NPAD, 1), x, W1)
    acc1 = _sc_agg_split(ei4, ew3, xwsa, xwsb, z32)
    hw2s = _tc2(acc1, xwsa, xwsb, dinv,
                b1.reshape(1, -1), W2)
    acc2 = _sc_agg32(ei4, ew3, hw2s, z32)
    z = _tc3(acc2, hw2s, dinv, b2.reshape(1, -1))
    return z


# speedup vs baseline: 1.0531x; 1.0531x over previous
import functools

import jax
import jax.numpy as jnp
from jax import lax
from jax.experimental import pallas as pl
from jax.experimental.pallas import tpu as pltpu
from jax.experimental.pallas import tpu_sc as plsc

N = 10000
E = 320000
DIN = 128
DOUT = 32
NPAD = 10240
NC = 2
NS = 16
NW = NC * NS
RPT = NPAD // NS
CH = 80
NCHUNK = 125
NBUF = 5
NBUF2 = 10
DEG_LAG = 8


def _sc_mesh():
    return plsc.VectorSubcoreMesh(core_axis_name="c", subcore_axis_name="s")


@functools.partial(
    pl.kernel,
    out_type=jax.ShapeDtypeStruct((NC, NPAD), jnp.float32),
    mesh=_sc_mesh(),
    scratch_types=[
        pltpu.VMEM((NCHUNK, CH), jnp.int32),
        pltpu.VMEM((NCHUNK, CH), jnp.float32),
        pltpu.VMEM_SHARED((NPAD,), jnp.float32),
        pltpu.SemaphoreType.DMA,
    ],
)
def _sc_deg(ei4, ew3, zeros, out, dst_all, ew_all, acc, ssem):
    cid = lax.axis_index("c")
    sid = lax.axis_index("s")
    wid = cid * NS + sid
    pltpu.sync_copy(ei4.at[1, wid], dst_all)
    pltpu.sync_copy(ew3.at[wid], ew_all)
    pltpu.sync_copy(zeros.at[pl.ds(sid * RPT, RPT)], acc.at[pl.ds(sid * RPT, RPT)])
    plsc.subcore_barrier()

    def fire(j):
        pltpu.make_async_copy(ew_all.at[j], acc.at[dst_all.at[j]], ssem).start(
            add=True
        )

    def drain(j):
        pltpu.make_async_copy(ew_all.at[j], acc.at[dst_all.at[j]], ssem).wait()

    def body(j, carry):
        fire(j)

        @pl.when(j >= DEG_LAG)
        def _():
            drain(j - DEG_LAG)

        return carry

    lax.fori_loop(0, NCHUNK, body, 0)
    for j in range(NCHUNK - DEG_LAG, NCHUNK):
        drain(j)
    plsc.subcore_barrier()
    pltpu.sync_copy(acc.at[pl.ds(sid * RPT, RPT)], out.at[cid, pl.ds(sid * RPT, RPT)])


def _make_sc_agg(D):
    @functools.partial(
        pl.kernel,
        out_type=jax.ShapeDtypeStruct((NC, NPAD, D), jnp.float32),
        mesh=_sc_mesh(),
        compiler_params=pltpu.CompilerParams(use_tc_tiling_on_sc=False),
        scratch_types=[
            pltpu.VMEM((NCHUNK, CH), jnp.int32),
            pltpu.VMEM((NCHUNK, CH), jnp.int32),
            pltpu.VMEM((NCHUNK, CH), jnp.float32),
        ]
        + [pltpu.VMEM((CH, D), jnp.float32) for _ in range(NBUF)]
        + [pltpu.VMEM_SHARED((NPAD, D), jnp.float32)]
        + [pltpu.SemaphoreType.DMA for _ in range(2 * NBUF)],
    )
    def _sc_agg(ei4, ew3, tbl, zeros, out, src_all, dst_all, ew_all,
                *bufs_acc_sems):
        rows = bufs_acc_sems[:NBUF]
        acc = bufs_acc_sems[NBUF]
        gsems = bufs_acc_sems[NBUF + 1:2 * NBUF + 1]
        ssems = bufs_acc_sems[2 * NBUF + 1:]
        cid = lax.axis_index("c")
        sid = lax.axis_index("s")
        wid = cid * NS + sid
        pltpu.sync_copy(ei4.at[0, wid], src_all)
        pltpu.sync_copy(ei4.at[1, wid], dst_all)
        pltpu.sync_copy(ew3.at[wid], ew_all)
        pltpu.sync_copy(zeros.at[pl.ds(sid * RPT, RPT)], acc.at[pl.ds(sid * RPT, RPT)])
        plsc.subcore_barrier()

        def g_copy(j, b):
            return pltpu.make_async_copy(tbl.at[src_all.at[j]], rows[b], gsems[b])

        def s_copy(j, b):
            return pltpu.make_async_copy(rows[b], acc.at[dst_all.at[j]], ssems[b])

        def scale(b, j):
            def grp(g, c):
                ev = ew_all[j, pl.ds(g * 16, 16)]
                for i in range(16):
                    s = ev[i]
                    for k in range(D // 16):
                        sl = pl.ds(k * 16, 16)
                        rows[b][g * 16 + i, sl] = rows[b][g * 16 + i, sl] * s
                return c

            lax.fori_loop(0, CH // 16, grp, 0)

        for b in range(NBUF):
            g_copy(b, b).start()

        def step(j2, carry):
            j0 = NBUF * j2
            for b in range(NBUF):
                j = j0 + b
                g_copy(j, b).wait()
                scale(b, j)
                s_copy(j, b).start(add=True)

                @pl.when(j + NBUF < NCHUNK)
                def _():
                    s_copy(j, b).wait()
                    g_copy(j + NBUF, b).start()

            return carry

        lax.fori_loop(0, NCHUNK // NBUF, step, 0)
        for b in range(NBUF):
            s_copy(NCHUNK - NBUF + b, b).wait()
        plsc.subcore_barrier()
        pltpu.sync_copy(
            acc.at[pl.ds(sid * RPT, RPT)], out.at[cid, pl.ds(sid * RPT, RPT)]
        )

    return _sc_agg


_sc_agg32 = _make_sc_agg(32)

_DS = 32


@functools.partial(
    pl.kernel,
    out_type=jax.ShapeDtypeStruct((NC, 2, NPAD, _DS), jnp.float32),
    mesh=_sc_mesh(),
    compiler_params=pltpu.CompilerParams(use_tc_tiling_on_sc=False),
    scratch_types=[
        pltpu.VMEM((NCHUNK, CH), jnp.int32),
        pltpu.VMEM((NCHUNK, CH), jnp.int32),
        pltpu.VMEM((NCHUNK, CH), jnp.float32),
    ]
    + [pltpu.VMEM((CH, _DS), jnp.float32) for _ in range(NBUF2)]
    + [
        pltpu.VMEM_SHARED((NPAD, _DS), jnp.float32),
        pltpu.VMEM_SHARED((NPAD, _DS), jnp.float32),
    ]
    + [pltpu.SemaphoreType.DMA for _ in range(2 * NBUF2)],
)
def _sc_agg_split(ei4, ew3, tbla, tblb, zeros, out, src_all, dst_all,
                  ew_all, *bufs_acc_sems):
    rows = bufs_acc_sems[:NBUF2]
    acca = bufs_acc_sems[NBUF2]
    accb = bufs_acc_sems[NBUF2 + 1]
    gsems = bufs_acc_sems[NBUF2 + 2:2 * NBUF2 + 2]
    ssems = bufs_acc_sems[2 * NBUF2 + 2:]
    tbls = (tbla, tblb)
    accs = (acca, accb)
    cid = lax.axis_index("c")
    sid = lax.axis_index("s")
    wid = cid * NS + sid
    pltpu.sync_copy(ei4.at[0, wid], src_all)
    pltpu.sync_copy(ei4.at[1, wid], dst_all)
    pltpu.sync_copy(ew3.at[wid], ew_all)
    sl_n = pl.ds(sid * RPT, RPT)
    pltpu.sync_copy(zeros.at[sl_n], acca.at[sl_n])
    pltpu.sync_copy(zeros.at[sl_n], accb.at[sl_n])
    plsc.subcore_barrier()

    def g_copy(j, b):
        return pltpu.make_async_copy(
            tbls[b & 1].at[src_all.at[j]], rows[b], gsems[b]
        )

    def s_copy(j, b):
        return pltpu.make_async_copy(
            rows[b], accs[b & 1].at[dst_all.at[j]], ssems[b]
        )

    def scale_pair(ba, bb, j):
        def grp(g, c):
            ev = ew_all[j, pl.ds(g * 16, 16)]
            for i in range(16):
                s = ev[i]
                for k in range(_DS // 16):
                    sl = pl.ds(k * 16, 16)
                    rows[ba][g * 16 + i, sl] = rows[ba][g * 16 + i, sl] * s
                    rows[bb][g * 16 + i, sl] = rows[bb][g * 16 + i, sl] * s
            return c

        lax.fori_loop(0, CH // 16, grp, 0)

    for b in range(NBUF2):
        g_copy(b >> 1, b).start()

    NT = 2 * NCHUNK

    def step(t2, carry):
        t0 = NBUF2 * t2
        for p in range(NBUF2 // 2):
            ba, bb = 2 * p, 2 * p + 1
            j = NBUF2 // 2 * t2 + p
            g_copy(j, ba).wait()
            g_copy(j, bb).wait()
            scale_pair(ba, bb, j)
            s_copy(j, ba).start(add=True)
            s_copy(j, bb).start(add=True)

            @pl.when(j + (NBUF2 >> 1) < NCHUNK)
            def _():
                s_copy(j, ba).wait()
                s_copy(j, bb).wait()
                g_copy(j + (NBUF2 >> 1), ba).start()
                g_copy(j + (NBUF2 >> 1), bb).start()

        return carry

    lax.fori_loop(0, NT // NBUF2, step, 0)
    for b in range(NBUF2):
        s_copy(NCHUNK - (NBUF2 >> 1) + (b >> 1), b).wait()
    plsc.subcore_barrier()
    pltpu.sync_copy(acca.at[sl_n], out.at[cid, 0, sl_n])
    pltpu.sync_copy(accb.at[sl_n], out.at[cid, 1, sl_n])


_GB = 10
_RB = N // _GB


def _tc1_body(degp, x, w, dinv_o, xwsa_o, xwsb_o):
    deg = degp[0, :N] + degp[1, :N] + 1.0
    dinv = lax.rsqrt(deg).reshape(N, 1)
    xw = jnp.dot(x[...].astype(jnp.bfloat16), w[...].astype(jnp.bfloat16),
                 preferred_element_type=jnp.float32)
    xws = xw * dinv
    dinv_o[...] = dinv
    xwsa_o[...] = xws[:, :_DS]
    xwsb_o[...] = xws[:, _DS:]


_tc1 = pl.pallas_call(
    _tc1_body,
    out_shape=[
        jax.ShapeDtypeStruct((N, 1), jnp.float32),
        jax.ShapeDtypeStruct((N, _DS), jnp.float32),
        jax.ShapeDtypeStruct((N, _DS), jnp.float32),
    ],
)


def _tc2_body(acc1, xwsa, xwsb, dinv, b1, w2, out):
    ta = (acc1[0, 0] + acc1[1, 0] + xwsa[...]) * dinv[...] + b1[:, :_DS]
    tb = (acc1[0, 1] + acc1[1, 1] + xwsb[...]) * dinv[...] + b1[:, _DS:]
    h = jnp.concatenate([jnp.maximum(ta, 0.0), jnp.maximum(tb, 0.0)], axis=1)
    hw2 = jnp.dot(h.astype(jnp.bfloat16), w2[...].astype(jnp.bfloat16),
                  preferred_element_type=jnp.float32)
    out[...] = hw2 * dinv[...]


_tc2 = pl.pallas_call(
    _tc2_body,
    grid=(_GB,),
    in_specs=[
        pl.BlockSpec((2, 2, _RB, _DS), lambda i: (0, 0, i, 0)),
        pl.BlockSpec((_RB, _DS), lambda i: (i, 0)),
        pl.BlockSpec((_RB, _DS), lambda i: (i, 0)),
        pl.BlockSpec((_RB, 1), lambda i: (i, 0)),
        pl.BlockSpec((1, 2 * _DS), lambda i: (0, 0)),
        pl.BlockSpec((2 * _DS, DOUT), lambda i: (0, 0)),
    ],
    out_specs=pl.BlockSpec((_RB, DOUT), lambda i: (i, 0)),
    out_shape=jax.ShapeDtypeStruct((N, DOUT), jnp.float32),
)


def _tc3_body(acc2, hw, dinv, b2, out):
    out[...] = (acc2[0, :N] + acc2[1, :N] + hw[...]) * dinv[...] + b2[...]


_tc3 = pl.pallas_call(
    _tc3_body,
    out_shape=jax.ShapeDtypeStruct((N, DOUT), jnp.float32),
)


def kernel(x, edge_index, edge_weight, W1, b1, W2, b2):
    ei4 = edge_index.astype(jnp.int32).reshape(2, NW, NCHUNK, CH)
    ew3 = edge_weight.reshape(NW, NCHUNK, CH)
    z1 = jnp.zeros((NPAD,), jnp.float32)
    z32 = jnp.zeros((NPAD, 32), jnp.float32)

    degp = _sc_deg(ei4, ew3, z1)
    dinv, xwsa, xwsb = _tc1(degp, x, W1)
    acc1 = _sc_agg_split(ei4, ew3, xwsa, xwsb, z32)
    hw2s = _tc2(acc1, xwsa, xwsb, dinv,
                b1.reshape(1, -1), W2)
    acc2 = _sc_agg32(ei4, ew3, hw2s, z32)
    z = _tc3(acc2, hw2s, dinv, b2.reshape(1, -1))
    return z


# speedup vs baseline: 1.0774x; 1.0231x over previous
import functools

import jax
import jax.numpy as jnp
from jax import lax
from jax.experimental import pallas as pl
from jax.experimental.pallas import tpu as pltpu
from jax.experimental.pallas import tpu_sc as plsc

N = 10000
E = 320000
DIN = 128
DOUT = 32
NPAD = 10240
NC = 2
NS = 16
NW = NC * NS
RPT = NPAD // NS
CH = 80
NCHUNK = 125
NBUF = 5
NBUF2 = 10
DEG_LAG = 8


def _sc_mesh():
    return plsc.VectorSubcoreMesh(core_axis_name="c", subcore_axis_name="s")


@functools.partial(
    pl.kernel,
    out_type=jax.ShapeDtypeStruct((NC, NPAD), jnp.float32),
    mesh=_sc_mesh(),
    compiler_params=pltpu.CompilerParams(use_tc_tiling_on_sc=False),
    scratch_types=[
        pltpu.VMEM((NCHUNK, CH), jnp.int32),
        pltpu.VMEM((NCHUNK, CH), jnp.float32),
        pltpu.VMEM_SHARED((NPAD,), jnp.float32),
        pltpu.SemaphoreType.DMA,
    ],
)
def _sc_deg(ei4, ew3, zeros, out, dst_all, ew_all, acc, ssem):
    cid = lax.axis_index("c")
    sid = lax.axis_index("s")
    wid = cid * NS + sid
    pltpu.sync_copy(ei4.at[1, wid], dst_all)
    pltpu.sync_copy(ew3.at[wid], ew_all)
    pltpu.sync_copy(zeros.at[pl.ds(sid * RPT, RPT)], acc.at[pl.ds(sid * RPT, RPT)])
    plsc.subcore_barrier()

    def fire(j):
        pltpu.make_async_copy(ew_all.at[j], acc.at[dst_all.at[j]], ssem).start(
            add=True
        )

    def drain(j):
        pltpu.make_async_copy(ew_all.at[j], acc.at[dst_all.at[j]], ssem).wait()

    def body(j, carry):
        fire(j)

        @pl.when(j >= DEG_LAG)
        def _():
            drain(j - DEG_LAG)

        return carry

    lax.fori_loop(0, NCHUNK, body, 0)
    for j in range(NCHUNK - DEG_LAG, NCHUNK):
        drain(j)
    plsc.subcore_barrier()
    pltpu.sync_copy(acc.at[pl.ds(sid * RPT, RPT)], out.at[cid, pl.ds(sid * RPT, RPT)])


def _make_sc_agg(D):
    @functools.partial(
        pl.kernel,
        out_type=jax.ShapeDtypeStruct((NC, NPAD, D), jnp.float32),
        mesh=_sc_mesh(),
        compiler_params=pltpu.CompilerParams(use_tc_tiling_on_sc=False),
        scratch_types=[
            pltpu.VMEM((NCHUNK, CH), jnp.int32),
            pltpu.VMEM((NCHUNK, CH), jnp.int32),
            pltpu.VMEM((NCHUNK, CH), jnp.float32),
        ]
        + [pltpu.VMEM((CH, D), jnp.float32) for _ in range(NBUF)]
        + [pltpu.VMEM_SHARED((NPAD, D), jnp.float32)]
        + [pltpu.SemaphoreType.DMA for _ in range(2 * NBUF)],
    )
    def _sc_agg(ei4, ew3, tbl, zeros, out, src_all, dst_all, ew_all,
                *bufs_acc_sems):
        rows = bufs_acc_sems[:NBUF]
        acc = bufs_acc_sems[NBUF]
        gsems = bufs_acc_sems[NBUF + 1:2 * NBUF + 1]
        ssems = bufs_acc_sems[2 * NBUF + 1:]
        cid = lax.axis_index("c")
        sid = lax.axis_index("s")
        wid = cid * NS + sid
        pltpu.sync_copy(ei4.at[0, wid], src_all)
        pltpu.sync_copy(ei4.at[1, wid], dst_all)
        pltpu.sync_copy(ew3.at[wid], ew_all)
        pltpu.sync_copy(zeros.at[pl.ds(sid * RPT, RPT)], acc.at[pl.ds(sid * RPT, RPT)])
        plsc.subcore_barrier()

        def g_copy(j, b):
            return pltpu.make_async_copy(tbl.at[src_all.at[j]], rows[b], gsems[b])

        def s_copy(j, b):
            return pltpu.make_async_copy(rows[b], acc.at[dst_all.at[j]], ssems[b])

        def scale(b, j):
            def grp(g, c):
                ev = ew_all[j, pl.ds(g * 16, 16)]
                for i in range(16):
                    s = ev[i]
                    for k in range(D // 16):
                        sl = pl.ds(k * 16, 16)
                        rows[b][g * 16 + i, sl] = rows[b][g * 16 + i, sl] * s
                return c

            lax.fori_loop(0, CH // 16, grp, 0)

        for b in range(NBUF):
            g_copy(b, b).start()

        def step(j2, carry):
            j0 = NBUF * j2
            for b in range(NBUF):
                j = j0 + b
                g_copy(j, b).wait()
                scale(b, j)
                s_copy(j, b).start(add=True)

                @pl.when(j + NBUF < NCHUNK)
                def _():
                    s_copy(j, b).wait()
                    g_copy(j + NBUF, b).start()

            return carry

        lax.fori_loop(0, NCHUNK // NBUF, step, 0)
        for b in range(NBUF):
            s_copy(NCHUNK - NBUF + b, b).wait()
        plsc.subcore_barrier()
        pltpu.sync_copy(
            acc.at[pl.ds(sid * RPT, RPT)], out.at[cid, pl.ds(sid * RPT, RPT)]
        )

    return _sc_agg


_sc_agg32 = _make_sc_agg(32)

_DS = 32


@functools.partial(
    pl.kernel,
    out_type=jax.ShapeDtypeStruct((NC, 2, NPAD, _DS), jnp.float32),
    mesh=_sc_mesh(),
    compiler_params=pltpu.CompilerParams(use_tc_tiling_on_sc=False),
    scratch_types=[
        pltpu.VMEM((NCHUNK, CH), jnp.int32),
        pltpu.VMEM((NCHUNK, CH), jnp.int32),
        pltpu.VMEM((NCHUNK, CH), jnp.float32),
    ]
    + [pltpu.VMEM((CH, _DS), jnp.float32) for _ in range(NBUF2)]
    + [
        pltpu.VMEM_SHARED((NPAD, _DS), jnp.float32),
        pltpu.VMEM_SHARED((NPAD, _DS), jnp.float32),
    ]
    + [pltpu.SemaphoreType.DMA for _ in range(2 * NBUF2)],
)
def _sc_agg_split(ei4, ew3, tbla, tblb, zeros, out, src_all, dst_all,
                  ew_all, *bufs_acc_sems):
    rows = bufs_acc_sems[:NBUF2]
    acca = bufs_acc_sems[NBUF2]
    accb = bufs_acc_sems[NBUF2 + 1]
    gsems = bufs_acc_sems[NBUF2 + 2:2 * NBUF2 + 2]
    ssems = bufs_acc_sems[2 * NBUF2 + 2:]
    tbls = (tbla, tblb)
    accs = (acca, accb)
    cid = lax.axis_index("c")
    sid = lax.axis_index("s")
    wid = cid * NS + sid
    pltpu.sync_copy(ei4.at[0, wid], src_all)
    pltpu.sync_copy(ei4.at[1, wid], dst_all)
    pltpu.sync_copy(ew3.at[wid], ew_all)
    sl_n = pl.ds(sid * RPT, RPT)
    pltpu.sync_copy(zeros.at[sl_n], acca.at[sl_n])
    pltpu.sync_copy(zeros.at[sl_n], accb.at[sl_n])
    plsc.subcore_barrier()

    def g_copy(j, b):
        return pltpu.make_async_copy(
            tbls[b & 1].at[src_all.at[j]], rows[b], gsems[b]
        )

    def s_copy(j, b):
        return pltpu.make_async_copy(
            rows[b], accs[b & 1].at[dst_all.at[j]], ssems[b]
        )

    def scale_pair(ba, bb, j):
        def grp(g, c):
            ev = ew_all[j, pl.ds(g * 16, 16)]
            for i in range(16):
                s = ev[i]
                for k in range(_DS // 16):
                    sl = pl.ds(k * 16, 16)
                    rows[ba][g * 16 + i, sl] = rows[ba][g * 16 + i, sl] * s
                    rows[bb][g * 16 + i, sl] = rows[bb][g * 16 + i, sl] * s
            return c

        lax.fori_loop(0, CH // 16, grp, 0)

    for b in range(NBUF2):
        g_copy(b >> 1, b).start()

    NT = 2 * NCHUNK

    def step(t2, carry):
        t0 = NBUF2 * t2
        for p in range(NBUF2 // 2):
            ba, bb = 2 * p, 2 * p + 1
            j = NBUF2 // 2 * t2 + p
            g_copy(j, ba).wait()
            g_copy(j, bb).wait()
            scale_pair(ba, bb, j)
            s_copy(j, ba).start(add=True)
            s_copy(j, bb).start(add=True)

            @pl.when(j + (NBUF2 >> 1) < NCHUNK)
            def _():
                s_copy(j, ba).wait()
                s_copy(j, bb).wait()
                g_copy(j + (NBUF2 >> 1), ba).start()
                g_copy(j + (NBUF2 >> 1), bb).start()

        return carry

    lax.fori_loop(0, NT // NBUF2, step, 0)
    for b in range(NBUF2):
        s_copy(NCHUNK - (NBUF2 >> 1) + (b >> 1), b).wait()
    plsc.subcore_barrier()
    pltpu.sync_copy(acca.at[sl_n], out.at[cid, 0, sl_n])
    pltpu.sync_copy(accb.at[sl_n], out.at[cid, 1, sl_n])


_GB = 10
_RB = N // _GB


def _tc1_body(degp, x, w, dinv_o, xwsa_o, xwsb_o):
    deg = degp[0, :N] + degp[1, :N] + 1.0
    dinv = lax.rsqrt(deg).reshape(N, 1)
    xw = jnp.dot(x[...].astype(jnp.bfloat16), w[...].astype(jnp.bfloat16),
                 preferred_element_type=jnp.float32)
    xws = xw * dinv
    dinv_o[...] = dinv
    xwsa_o[...] = xws[:, :_DS]
    xwsb_o[...] = xws[:, _DS:]


_tc1 = pl.pallas_call(
    _tc1_body,
    out_shape=[
        jax.ShapeDtypeStruct((N, 1), jnp.float32),
        jax.ShapeDtypeStruct((N, _DS), jnp.float32),
        jax.ShapeDtypeStruct((N, _DS), jnp.float32),
    ],
)


def _tc2_body(acc1, xwsa, xwsb, dinv, b1, w2, out):
    ta = (acc1[0, 0] + acc1[1, 0] + xwsa[...]) * dinv[...] + b1[:, :_DS]
    tb = (acc1[0, 1] + acc1[1, 1] + xwsb[...]) * dinv[...] + b1[:, _DS:]
    h = jnp.concatenate([jnp.maximum(ta, 0.0), jnp.maximum(tb, 0.0)], axis=1)
    hw2 = jnp.dot(h.astype(jnp.bfloat16), w2[...].astype(jnp.bfloat16),
                  preferred_element_type=jnp.float32)
    out[...] = hw2 * dinv[...]


_tc2 = pl.pallas_call(
    _tc2_body,
    grid=(_GB,),
    in_specs=[
        pl.BlockSpec((2, 2, _RB, _DS), lambda i: (0, 0, i, 0)),
        pl.BlockSpec((_RB, _DS), lambda i: (i, 0)),
        pl.BlockSpec((_RB, _DS), lambda i: (i, 0)),
        pl.BlockSpec((_RB, 1), lambda i: (i, 0)),
        pl.BlockSpec((1, 2 * _DS), lambda i: (0, 0)),
        pl.BlockSpec((2 * _DS, DOUT), lambda i: (0, 0)),
    ],
    out_specs=pl.BlockSpec((_RB, DOUT), lambda i: (i, 0)),
    out_shape=jax.ShapeDtypeStruct((N, DOUT), jnp.float32),
)


def _tc3_body(acc2, hw, dinv, b2, out):
    out[...] = (acc2[0, :N] + acc2[1, :N] + hw[...]) * dinv[...] + b2[...]


_tc3 = pl.pallas_call(
    _tc3_body,
    out_shape=jax.ShapeDtypeStruct((N, DOUT), jnp.float32),
)


def kernel(x, edge_index, edge_weight, W1, b1, W2, b2):
    ei4 = edge_index.astype(jnp.int32).reshape(2, NW, NCHUNK, CH)
    ew3 = edge_weight.reshape(NW, NCHUNK, CH)
    z1 = jnp.zeros((NPAD,), jnp.float32)
    z32 = jnp.zeros((NPAD, 32), jnp.float32)

    degp = _sc_deg(ei4, ew3, z1)
    dinv, xwsa, xwsb = _tc1(degp, x, W1)
    acc1 = _sc_agg_split(ei4, ew3, xwsa, xwsb, z32)
    hw2s = _tc2(acc1, xwsa, xwsb, dinv,
                b1.reshape(1, -1), W2)
    acc2 = _sc_agg32(ei4, ew3, hw2s, z32)
    z = _tc3(acc2, hw2s, dinv, b2.reshape(1, -1))
    return z
